# baseline (speedup 1.0000x reference)
import functools

import jax
import jax.numpy as jnp
from jax import lax
from jax.experimental import pallas as pl
from jax.experimental.pallas import tpu as pltpu
from jax.experimental.pallas import tpu_sc as plsc

N = 10000
E = 320000
D = 128
HD = D // 2
G = 64

NUM_CORES = 2
NUM_SUBCORES = 16
CHUNK = 128
E_PAD = 327680
PAD = E_PAD - E
EDGES_PER_SUBCORE = E_PAD // NUM_SUBCORES
NCHUNKS = EDGES_PER_SUBCORE // CHUNK
NSLOTS = 5
NSTEPS = NCHUNKS // NSLOTS
ROWS_PER_SUBCORE = 624
TAIL_ROWS = N - NUM_SUBCORES * ROWS_PER_SUBCORE


def _sc_segment_sum(src2, dst2, hs):
    mesh = plsc.VectorSubcoreMesh(
        core_axis_name="c", subcore_axis_name="s",
        num_cores=NUM_CORES, num_subcores=NUM_SUBCORES)

    @functools.partial(
        pl.kernel,
        out_type=jax.ShapeDtypeStruct((NUM_CORES, N, HD), jnp.float32),
        mesh=mesh,
        scratch_types=(
            [pltpu.VMEM((CHUNK,), jnp.int32)] * NSLOTS
            + [pltpu.VMEM((CHUNK,), jnp.int32)] * NSLOTS
            + [pltpu.VMEM((CHUNK, HD), jnp.float32)] * NSLOTS
            + [pltpu.VMEM((ROWS_PER_SUBCORE, HD), jnp.float32)]
            + [pltpu.VMEM_SHARED((N, HD), jnp.float32)]
            + [pltpu.SemaphoreType.DMA] * (3 * NSLOTS)
        ),
        compiler_params=pltpu.CompilerParams(use_tc_tiling_on_sc=False),
    )
    def k(src_hbm, dst_hbm, hs_hbm, out_hbm, *refs):
        src_bufs = refs[0:NSLOTS]
        dst_bufs = refs[NSLOTS:2 * NSLOTS]
        row_bufs = refs[2 * NSLOTS:3 * NSLOTS]
        zero_v = refs[3 * NSLOTS]
        agg_sh = refs[3 * NSLOTS + 1]
        isems = refs[3 * NSLOTS + 2:3 * NSLOTS + 2 + NSLOTS]
        gsems = refs[3 * NSLOTS + 2 + NSLOTS:3 * NSLOTS + 2 + 2 * NSLOTS]
        ssems = refs[3 * NSLOTS + 2 + 2 * NSLOTS:3 * NSLOTS + 2 + 3 * NSLOTS]
        cid = lax.axis_index("c")
        sid = lax.axis_index("s")

        zvec = jnp.zeros((16,), jnp.float32)

        def zrow(r, carry):
            for j in range(HD // 16):
                zero_v[r, pl.ds(j * 16, 16)] = zvec
            return carry

        lax.fori_loop(0, ROWS_PER_SUBCORE, zrow, None)

        row0 = sid * ROWS_PER_SUBCORE
        pltpu.sync_copy(zero_v, agg_sh.at[pl.ds(row0, ROWS_PER_SUBCORE)])

        @pl.when(sid == NUM_SUBCORES - 1)
        def _():
            pltpu.sync_copy(zero_v.at[pl.ds(0, TAIL_ROWS)],
                            agg_sh.at[pl.ds(N - TAIL_ROWS, TAIL_ROWS)])

        plsc.subcore_barrier()

        ebase = sid * EDGES_PER_SUBCORE
        h_view = hs_hbm.at[cid]

        def step(t, carry):
            off = ebase + t * NSLOTS * CHUNK
            iloads = []
            for j in range(NSLOTS):
                o = off + j * CHUNK
                i0 = pltpu.async_copy(src_hbm.at[pl.ds(o, CHUNK)],
                                      src_bufs[j], isems[j])
                i1 = pltpu.async_copy(dst_hbm.at[pl.ds(o, CHUNK)],
                                      dst_bufs[j], isems[j])
                iloads.append((i0, i1))
            gathers = []
            for j in range(NSLOTS):
                iloads[j][0].wait()
                iloads[j][1].wait()
                gathers.append(pltpu.async_copy(h_view.at[src_bufs[j]],
                                                row_bufs[j], gsems[j]))
            scatters = []
            for j in range(NSLOTS):
                gathers[j].wait()
                scatters.append(pltpu.async_copy(
                    row_bufs[j], agg_sh.at[dst_bufs[j]], ssems[j], add=True))
            for s in scatters:
                s.wait()
            return carry

        lax.fori_loop(0, NSTEPS, step, None)
        plsc.subcore_barrier()

        pltpu.sync_copy(agg_sh.at[pl.ds(row0, ROWS_PER_SUBCORE)],
                        out_hbm.at[cid, pl.ds(row0, ROWS_PER_SUBCORE)])

        @pl.when(sid == NUM_SUBCORES - 1)
        def _():
            pltpu.sync_copy(agg_sh.at[pl.ds(N - TAIL_ROWS, TAIL_ROWS)],
                            out_hbm.at[cid, pl.ds(N - TAIL_ROWS, TAIL_ROWS)])

    return k(src2, dst2, hs)


_VMEM_SPEC = pl.BlockSpec(memory_space=pltpu.VMEM)
_SMEM_SPEC = pl.BlockSpec(memory_space=pltpu.SMEM)


def _dense_layer(hs, parts, W, b, scale, g, be):
    def body(hs_ref, parts_ref, W_ref, b_ref, sc_ref, g_ref, be_ref, o_ref):
        h = jnp.concatenate([hs_ref[0], hs_ref[1]], axis=1)
        agg = jnp.concatenate([parts_ref[0], parts_ref[1]], axis=1)
        row0 = (lax.broadcasted_iota(jnp.int32, (N, 1), 0) == 0)
        agg = agg - jnp.where(row0, jnp.float32(PAD), 0.0) * h[0:1, :]
        y = sc_ref[0, 0] * h + agg
        y = jnp.dot(y, W_ref[...], preferred_element_type=jnp.float32)
        y = y + b_ref[...]
        mu = jnp.mean(y, axis=0, keepdims=True)
        var = jnp.mean((y - mu) ** 2, axis=0, keepdims=True)
        y = (y - mu) * lax.rsqrt(var + 1e-5) * g_ref[...] + be_ref[...]
        y = jnp.maximum(y, 0.0)
        o_ref[0] = y[:, :HD]
        o_ref[1] = y[:, HD:]

    return pl.pallas_call(
        body,
        out_shape=jax.ShapeDtypeStruct((NUM_CORES, N, HD), jnp.float32),
        in_specs=[_VMEM_SPEC, _VMEM_SPEC, _VMEM_SPEC, _VMEM_SPEC, _SMEM_SPEC,
                  _VMEM_SPEC, _VMEM_SPEC],
        out_specs=_VMEM_SPEC,
    )(hs, parts, W, b, scale, g, be)


def _pool(hs, batch):
    def body(hs_ref, batch_ref, o_ref):
        h = jnp.concatenate([hs_ref[0], hs_ref[1]], axis=1)
        seg = lax.broadcasted_iota(jnp.int32, (G, N), 0)
        onehot = (seg == batch_ref[...].reshape(1, N)).astype(jnp.float32)
        sums = jnp.dot(onehot, h, preferred_element_type=jnp.float32)
        counts = jnp.sum(onehot, axis=1, keepdims=True)
        o_ref[...] = sums / jnp.maximum(counts, 1.0)

    return pl.pallas_call(
        body,
        out_shape=jax.ShapeDtypeStruct((G, D), jnp.float32),
        in_specs=[_VMEM_SPEC, _VMEM_SPEC],
        out_specs=_VMEM_SPEC,
    )(hs, batch)


def kernel(edge_index, x, batch, W0, b0, eps0, g0, be0, W1, b1, eps1, g1, be1,
           W2, b2, eps2, g2, be2):
    zpad = jnp.zeros((PAD,), jnp.int32)
    src2 = jnp.concatenate([edge_index[0], zpad])
    dst2 = jnp.concatenate([edge_index[1], zpad])
    batch2d = batch.reshape(N, 1)
    hs = jnp.stack([x[:, :HD], x[:, HD:]])
    stacked = dict(
        W=jnp.stack([W0, W1, W2]),
        b=jnp.stack([b0, b1, b2]).reshape(3, 1, D),
        scale=(1.0 + jnp.stack([eps0, eps1, eps2])).reshape(3, 1, 1),
        g=jnp.stack([g0, g1, g2]).reshape(3, 1, D),
        be=jnp.stack([be0, be1, be2]).reshape(3, 1, D),
    )

    def layer(carry, p):
        parts = _sc_segment_sum(src2, dst2, carry)
        carry = _dense_layer(carry, parts, p["W"], p["b"], p["scale"],
                             p["g"], p["be"])
        return carry, None

    hs3, _ = lax.scan(layer, hs, stacked)
    return _pool(hs3, batch2d)


# speedup vs baseline: 1.9271x; 1.9271x over previous
import functools

import jax
import jax.numpy as jnp
from jax import lax
from jax.experimental import pallas as pl
from jax.experimental.pallas import tpu as pltpu
from jax.experimental.pallas import tpu_sc as plsc

N = 10000
E = 320000
D = 128
HD = D // 2
G = 64

NUM_CORES = 2
NUM_SUBCORES = 16
CHUNK = 128
E_PAD = 327680
PAD = E_PAD - E
EDGES_PER_SUBCORE = E_PAD // NUM_SUBCORES
NCHUNKS = EDGES_PER_SUBCORE // CHUNK
NSLOTS = 5
NSTEPS = NCHUNKS // NSLOTS
ROWS_PER_SUBCORE = 624
TAIL_ROWS = N - NUM_SUBCORES * ROWS_PER_SUBCORE


def _sc_segment_sum(src2, dst2, hs):
    mesh = plsc.VectorSubcoreMesh(
        core_axis_name="c", subcore_axis_name="s",
        num_cores=NUM_CORES, num_subcores=NUM_SUBCORES)

    @functools.partial(
        pl.kernel,
        out_type=jax.ShapeDtypeStruct((NUM_CORES, N, HD), jnp.float32),
        mesh=mesh,
        scratch_types=(
            [pltpu.VMEM((CHUNK,), jnp.int32)] * NSLOTS
            + [pltpu.VMEM((CHUNK,), jnp.int32)] * NSLOTS
            + [pltpu.VMEM((CHUNK, HD), jnp.float32)] * NSLOTS
            + [pltpu.VMEM((ROWS_PER_SUBCORE, HD), jnp.float32)]
            + [pltpu.VMEM_SHARED((N, HD), jnp.float32)]
            + [pltpu.SemaphoreType.DMA] * (3 * NSLOTS)
        ),
        compiler_params=pltpu.CompilerParams(use_tc_tiling_on_sc=False),
    )
    def k(src_hbm, dst_hbm, hs_hbm, out_hbm, *refs):
        src_bufs = refs[0:NSLOTS]
        dst_bufs = refs[NSLOTS:2 * NSLOTS]
        row_bufs = refs[2 * NSLOTS:3 * NSLOTS]
        zero_v = refs[3 * NSLOTS]
        agg_sh = refs[3 * NSLOTS + 1]
        isems = refs[3 * NSLOTS + 2:3 * NSLOTS + 2 + NSLOTS]
        gsems = refs[3 * NSLOTS + 2 + NSLOTS:3 * NSLOTS + 2 + 2 * NSLOTS]
        ssems = refs[3 * NSLOTS + 2 + 2 * NSLOTS:3 * NSLOTS + 2 + 3 * NSLOTS]
        cid = lax.axis_index("c")
        sid = lax.axis_index("s")

        zvec = jnp.zeros((16,), jnp.float32)

        def zrow(r, carry):
            for j in range(HD // 16):
                zero_v[r, pl.ds(j * 16, 16)] = zvec
            return carry

        lax.fori_loop(0, ROWS_PER_SUBCORE, zrow, None)

        row0 = sid * ROWS_PER_SUBCORE
        pltpu.sync_copy(zero_v, agg_sh.at[pl.ds(row0, ROWS_PER_SUBCORE)])

        @pl.when(sid == NUM_SUBCORES - 1)
        def _():
            pltpu.sync_copy(zero_v.at[pl.ds(0, TAIL_ROWS)],
                            agg_sh.at[pl.ds(N - TAIL_ROWS, TAIL_ROWS)])

        plsc.subcore_barrier()

        ebase = sid * EDGES_PER_SUBCORE
        h_view = hs_hbm.at[cid]

        def step(t, carry):
            off = ebase + t * NSLOTS * CHUNK
            iloads = []
            for j in range(NSLOTS):
                o = off + j * CHUNK
                i0 = pltpu.async_copy(src_hbm.at[pl.ds(o, CHUNK)],
                                      src_bufs[j], isems[j])
                i1 = pltpu.async_copy(dst_hbm.at[pl.ds(o, CHUNK)],
                                      dst_bufs[j], isems[j])
                iloads.append((i0, i1))
            gathers = []
            for j in range(NSLOTS):
                iloads[j][0].wait()
                iloads[j][1].wait()
                gathers.append(pltpu.async_copy(h_view.at[src_bufs[j]],
                                                row_bufs[j], gsems[j]))
            scatters = []
            for j in range(NSLOTS):
                gathers[j].wait()
                scatters.append(pltpu.async_copy(
                    row_bufs[j], agg_sh.at[dst_bufs[j]], ssems[j], add=True))
            for s in scatters:
                s.wait()
            return carry

        lax.fori_loop(0, NSTEPS, step, None)
        plsc.subcore_barrier()

        pltpu.sync_copy(agg_sh.at[pl.ds(row0, ROWS_PER_SUBCORE)],
                        out_hbm.at[cid, pl.ds(row0, ROWS_PER_SUBCORE)])

        @pl.when(sid == NUM_SUBCORES - 1)
        def _():
            pltpu.sync_copy(agg_sh.at[pl.ds(N - TAIL_ROWS, TAIL_ROWS)],
                            out_hbm.at[cid, pl.ds(N - TAIL_ROWS, TAIL_ROWS)])

    return k(src2, dst2, hs)


_VMEM_SPEC = pl.BlockSpec(memory_space=pltpu.VMEM)
_SMEM_SPEC = pl.BlockSpec(memory_space=pltpu.SMEM)


def _dense_layer(hs, parts, W, b, scale, g, be):
    def body(hs_ref, parts_ref, W_ref, b_ref, sc_ref, g_ref, be_ref, o_ref):
        h = jnp.concatenate([hs_ref[0], hs_ref[1]], axis=1)
        agg = jnp.concatenate([parts_ref[0], parts_ref[1]], axis=1)
        padrow = (lax.broadcasted_iota(jnp.int32, (N, 1), 0) < PAD)
        agg = agg - jnp.where(padrow, 1.0, 0.0) * h
        y = sc_ref[0, 0] * h + agg
        y = jnp.dot(y, W_ref[...], preferred_element_type=jnp.float32)
        y = y + b_ref[...]
        mu = jnp.mean(y, axis=0, keepdims=True)
        var = jnp.mean((y - mu) ** 2, axis=0, keepdims=True)
        y = (y - mu) * lax.rsqrt(var + 1e-5) * g_ref[...] + be_ref[...]
        y = jnp.maximum(y, 0.0)
        o_ref[0] = y[:, :HD]
        o_ref[1] = y[:, HD:]

    return pl.pallas_call(
        body,
        out_shape=jax.ShapeDtypeStruct((NUM_CORES, N, HD), jnp.float32),
        in_specs=[_VMEM_SPEC, _VMEM_SPEC, _VMEM_SPEC, _VMEM_SPEC, _SMEM_SPEC,
                  _VMEM_SPEC, _VMEM_SPEC],
        out_specs=_VMEM_SPEC,
    )(hs, parts, W, b, scale, g, be)


def _pool(hs, batch):
    def body(hs_ref, batch_ref, o_ref):
        h = jnp.concatenate([hs_ref[0], hs_ref[1]], axis=1)
        seg = lax.broadcasted_iota(jnp.int32, (G, N), 0)
        onehot = (seg == batch_ref[...].reshape(1, N)).astype(jnp.float32)
        sums = jnp.dot(onehot, h, preferred_element_type=jnp.float32)
        counts = jnp.sum(onehot, axis=1, keepdims=True)
        o_ref[...] = sums / jnp.maximum(counts, 1.0)

    return pl.pallas_call(
        body,
        out_shape=jax.ShapeDtypeStruct((G, D), jnp.float32),
        in_specs=[_VMEM_SPEC, _VMEM_SPEC],
        out_specs=_VMEM_SPEC,
    )(hs, batch)


def kernel(edge_index, x, batch, W0, b0, eps0, g0, be0, W1, b1, eps1, g1, be1,
           W2, b2, eps2, g2, be2):
    zpad = jnp.arange(PAD, dtype=jnp.int32)
    src2 = jnp.concatenate([edge_index[0], zpad])
    dst2 = jnp.concatenate([edge_index[1], zpad])
    batch2d = batch.reshape(N, 1)
    hs = jnp.stack([x[:, :HD], x[:, HD:]])
    stacked = dict(
        W=jnp.stack([W0, W1, W2]),
        b=jnp.stack([b0, b1, b2]).reshape(3, 1, D),
        scale=(1.0 + jnp.stack([eps0, eps1, eps2])).reshape(3, 1, 1),
        g=jnp.stack([g0, g1, g2]).reshape(3, 1, D),
        be=jnp.stack([be0, be1, be2]).reshape(3, 1, D),
    )

    def layer(carry, p):
        parts = _sc_segment_sum(src2, dst2, carry)
        carry = _dense_layer(carry, parts, p["W"], p["b"], p["scale"],
                             p["g"], p["be"])
        return carry, None

    hs3, _ = lax.scan(layer, hs, stacked)
    return _pool(hs3, batch2d)


# speedup vs baseline: 2.0729x; 1.0757x over previous
import functools

import jax
import jax.numpy as jnp
from jax import lax
from jax.experimental import pallas as pl
from jax.experimental.pallas import tpu as pltpu
from jax.experimental.pallas import tpu_sc as plsc

N = 10000
E = 320000
D = 128
HD = D // 2
G = 64

NUM_CORES = 2
NUM_SUBCORES = 16
CHUNK = 128
E_PAD = 327680
PAD = E_PAD - E
EDGES_PER_SUBCORE = E_PAD // NUM_SUBCORES
NCHUNKS = EDGES_PER_SUBCORE // CHUNK
NSLOTS = 5
NSTEPS = NCHUNKS // NSLOTS
ROWS_PER_SUBCORE = 624
TAIL_ROWS = N - NUM_SUBCORES * ROWS_PER_SUBCORE


def _sc_segment_sum(src2, dst2, hs):
    mesh = plsc.VectorSubcoreMesh(
        core_axis_name="c", subcore_axis_name="s",
        num_cores=NUM_CORES, num_subcores=NUM_SUBCORES)

    @functools.partial(
        pl.kernel,
        out_type=jax.ShapeDtypeStruct((NUM_CORES, N, HD), jnp.float32),
        mesh=mesh,
        scratch_types=(
            [pltpu.VMEM((CHUNK,), jnp.int32)] * NSLOTS
            + [pltpu.VMEM((CHUNK,), jnp.int32)] * NSLOTS
            + [pltpu.VMEM((CHUNK, HD), jnp.float32)] * NSLOTS
            + [pltpu.VMEM((ROWS_PER_SUBCORE, HD), jnp.float32)]
            + [pltpu.VMEM_SHARED((N, HD), jnp.float32)]
            + [pltpu.SemaphoreType.DMA] * (3 * NSLOTS)
        ),
        compiler_params=pltpu.CompilerParams(use_tc_tiling_on_sc=False),
    )
    def k(src_hbm, dst_hbm, hs_hbm, out_hbm, *refs):
        src_bufs = refs[0:NSLOTS]
        dst_bufs = refs[NSLOTS:2 * NSLOTS]
        row_bufs = refs[2 * NSLOTS:3 * NSLOTS]
        zero_v = refs[3 * NSLOTS]
        agg_sh = refs[3 * NSLOTS + 1]
        isems = refs[3 * NSLOTS + 2:3 * NSLOTS + 2 + NSLOTS]
        gsems = refs[3 * NSLOTS + 2 + NSLOTS:3 * NSLOTS + 2 + 2 * NSLOTS]
        ssems = refs[3 * NSLOTS + 2 + 2 * NSLOTS:3 * NSLOTS + 2 + 3 * NSLOTS]
        cid = lax.axis_index("c")
        sid = lax.axis_index("s")

        zvec = jnp.zeros((16,), jnp.float32)

        def zrow(r, carry):
            for j in range(HD // 16):
                zero_v[r, pl.ds(j * 16, 16)] = zvec
            return carry

        lax.fori_loop(0, ROWS_PER_SUBCORE, zrow, None)

        row0 = sid * ROWS_PER_SUBCORE
        pltpu.sync_copy(zero_v, agg_sh.at[pl.ds(row0, ROWS_PER_SUBCORE)])

        @pl.when(sid == NUM_SUBCORES - 1)
        def _():
            pltpu.sync_copy(zero_v.at[pl.ds(0, TAIL_ROWS)],
                            agg_sh.at[pl.ds(N - TAIL_ROWS, TAIL_ROWS)])

        plsc.subcore_barrier()

        ebase = sid * EDGES_PER_SUBCORE
        h_view = hs_hbm.at[cid]

        def step(t, carry):
            off = ebase + t * NSLOTS * CHUNK
            iloads = []
            for j in range(NSLOTS):
                o = off + j * CHUNK
                i0 = pltpu.async_copy(src_hbm.at[pl.ds(o, CHUNK)],
                                      src_bufs[j], isems[j])
                i1 = pltpu.async_copy(dst_hbm.at[pl.ds(o, CHUNK)],
                                      dst_bufs[j], isems[j])
                iloads.append((i0, i1))
            gathers = []
            for j in range(NSLOTS):
                iloads[j][0].wait()
                iloads[j][1].wait()
                gathers.append(pltpu.async_copy(h_view.at[src_bufs[j]],
                                                row_bufs[j], gsems[j]))
            scatters = []
            for j in range(NSLOTS):
                gathers[j].wait()
                scatters.append(pltpu.async_copy(
                    row_bufs[j], agg_sh.at[dst_bufs[j]], ssems[j], add=True))
            for s in scatters:
                s.wait()
            return carry

        lax.fori_loop(0, NSTEPS, step, None)
        plsc.subcore_barrier()

        pltpu.sync_copy(agg_sh.at[pl.ds(row0, ROWS_PER_SUBCORE)],
                        out_hbm.at[cid, pl.ds(row0, ROWS_PER_SUBCORE)])

        @pl.when(sid == NUM_SUBCORES - 1)
        def _():
            pltpu.sync_copy(agg_sh.at[pl.ds(N - TAIL_ROWS, TAIL_ROWS)],
                            out_hbm.at[cid, pl.ds(N - TAIL_ROWS, TAIL_ROWS)])

    return k(src2, dst2, hs)


_VMEM_SPEC = pl.BlockSpec(memory_space=pltpu.VMEM)
_SMEM_SPEC = pl.BlockSpec(memory_space=pltpu.SMEM)


def _dense_common(hs_ref, parts_ref, W_ref, b_ref, sc_ref, g_ref, be_ref):
    h = jnp.concatenate([hs_ref[0], hs_ref[1]], axis=1)
    agg = jnp.concatenate([parts_ref[0], parts_ref[1]], axis=1)
    padrow = (lax.broadcasted_iota(jnp.int32, (N, 1), 0) < PAD)
    agg = agg - jnp.where(padrow, 1.0, 0.0) * h
    y = sc_ref[0, 0] * h + agg
    y = jnp.dot(y, W_ref[...], preferred_element_type=jnp.float32)
    y = y + b_ref[...]
    mu = jnp.mean(y, axis=0, keepdims=True)
    var = jnp.mean((y - mu) ** 2, axis=0, keepdims=True)
    y = (y - mu) * lax.rsqrt(var + 1e-5) * g_ref[...] + be_ref[...]
    return jnp.maximum(y, 0.0)


def _dense_layer(hs, parts, W, b, scale, g, be):
    def body(hs_ref, parts_ref, W_ref, b_ref, sc_ref, g_ref, be_ref, o_ref):
        y = _dense_common(hs_ref, parts_ref, W_ref, b_ref, sc_ref, g_ref,
                          be_ref)
        o_ref[0] = y[:, :HD]
        o_ref[1] = y[:, HD:]

    return pl.pallas_call(
        body,
        out_shape=jax.ShapeDtypeStruct((NUM_CORES, N, HD), jnp.float32),
        in_specs=[_VMEM_SPEC, _VMEM_SPEC, _VMEM_SPEC, _VMEM_SPEC, _SMEM_SPEC,
                  _VMEM_SPEC, _VMEM_SPEC],
        out_specs=_VMEM_SPEC,
    )(hs, parts, W, b, scale, g, be)


def _dense_layer_pool(hs, parts, W, b, scale, g, be, batch):
    def body(hs_ref, parts_ref, W_ref, b_ref, sc_ref, g_ref, be_ref,
             batch_ref, o_ref):
        y = _dense_common(hs_ref, parts_ref, W_ref, b_ref, sc_ref, g_ref,
                          be_ref)
        seg = lax.broadcasted_iota(jnp.int32, (G, N), 0)
        onehot = (seg == batch_ref[...].reshape(1, N)).astype(jnp.float32)
        sums = jnp.dot(onehot, y, preferred_element_type=jnp.float32)
        counts = jnp.sum(onehot, axis=1, keepdims=True)
        o_ref[...] = sums / jnp.maximum(counts, 1.0)

    return pl.pallas_call(
        body,
        out_shape=jax.ShapeDtypeStruct((G, D), jnp.float32),
        in_specs=[_VMEM_SPEC, _VMEM_SPEC, _VMEM_SPEC, _VMEM_SPEC, _SMEM_SPEC,
                  _VMEM_SPEC, _VMEM_SPEC, _VMEM_SPEC],
        out_specs=_VMEM_SPEC,
    )(hs, parts, W, b, scale, g, be, batch)


def kernel(edge_index, x, batch, W0, b0, eps0, g0, be0, W1, b1, eps1, g1, be1,
           W2, b2, eps2, g2, be2):
    zpad = jnp.arange(PAD, dtype=jnp.int32)
    src2 = jnp.concatenate([edge_index[0], zpad])
    dst2 = jnp.concatenate([edge_index[1], zpad])
    batch2d = batch.reshape(N, 1)
    hs = jnp.stack([x[:, :HD], x[:, HD:]])
    params = [(W0, b0, eps0, g0, be0), (W1, b1, eps1, g1, be1),
              (W2, b2, eps2, g2, be2)]
    for i, (W, b, eps, g, be) in enumerate(params):
        parts = _sc_segment_sum(src2, dst2, hs)
        scale = (1.0 + eps).reshape(1, 1)
        b2d = b.reshape(1, D)
        g2d = g.reshape(1, D)
        be2d = be.reshape(1, D)
        if i < 2:
            hs = _dense_layer(hs, parts, W, b2d, scale, g2d, be2d)
        else:
            out = _dense_layer_pool(hs, parts, W, b2d, scale, g2d, be2d,
                                    batch2d)
    return out
